# baseline (speedup 1.0000x reference)
import jax
import jax.numpy as jnp
from jax.experimental import pallas as pl


def _bn_(x):
    axes = tuple(range(x.ndim - 1))
    m = jnp.mean(x, axis=axes, keepdims=True)
    v = jnp.var(x, axis=axes, keepdims=True)
    return (x - m) / jnp.sqrt(v + 1e-5)


def _knn_(x, k):
    sq = jnp.sum(x * x, axis=1)
    d = sq[:, None] - 2.0 * (x @ x.T) + sq[None, :]
    _, idx = jax.lax.top_k(-d, k)
    return idx


def _edge_conv_(x, W, k):
    idx = jax.vmap(lambda xi: _knn_(xi, k))(x)
    nb = jax.vmap(lambda xi, ii: xi[ii])(x, idx)
    center = x[:, :, None, :]
    feat = jnp.concatenate([nb - center, jnp.broadcast_to(center, nb.shape)], axis=-1)
    h = jax.nn.leaky_relu(_bn_(feat @ W), 0.2)
    return jnp.max(h, axis=2)


def _sa_(xyz, W1, W2, ns):
    idx = jax.vmap(lambda xi: _knn_(xi, ns))(xyz)
    nb = jax.vmap(lambda xi, ii: xi[ii])(xyz, idx)
    rel = nb - xyz[:, :, None, :]
    h = jax.nn.relu(_bn_(rel @ W1))
    h = jax.nn.relu(_bn_(h @ W2))
    return jnp.max(h, axis=2)


def _eig_feats_(xyz_b):
    N = xyz_b.shape[0]
    d2 = jnp.sum((xyz_b[:, None, :] - xyz_b[None, :, :]) ** 2, axis=-1)
    d = jnp.sqrt(d2 + 1e-12)
    eye = jnp.eye(N, dtype=bool)
    d_inf = jnp.where(eye, jnp.inf, d)
    max_d = jnp.max(jnp.where(eye, -jnp.inf, d))
    radius = max_d * 0.1
    mask = d_inf < radius
    jitter = 1e-6 * jnp.diag(jnp.array([1.0, 2.0, 3.0], dtype=xyz_b.dtype))
    def per_point(mi):
        cnt = jnp.maximum(jnp.sum(mi), 1).astype(xyz_b.dtype)
        m = mi[:, None].astype(xyz_b.dtype)
        mean = jnp.sum(xyz_b * m, axis=0) / cnt
        centered = (xyz_b - mean) * m
        cov = centered.T @ centered / N + jitter
        return jnp.linalg.eigvalsh(cov)
    return jax.vmap(per_point)(mask)


def kernel(pointcloud, W_sa1, W_sa2, Wd1, Wd2, Wd3, Wd4, Wd5, Wg1, Wg2, Wg3,
           We1, be1, We2, be2, Wc1, Wc2, Wc3, numpoints):
    xyz = pointcloud[..., 0:3]
    h1 = _sa_(xyz, W_sa1, W_sa2, 32)
    x1 = _edge_conv_(xyz, Wd1, 20)
    x2 = _edge_conv_(x1, Wd2, 20)
    x3 = _edge_conv_(x2, Wd3, 20)
    x4 = _edge_conv_(x3, Wd4, 20)
    xc = jnp.concatenate([x1, x2, x3, x4], axis=-1)
    h2 = jax.nn.leaky_relu(_bn_(xc @ Wd5), 0.2)
    h2 = jax.nn.relu(_bn_(h2 @ Wg1))
    h2 = jax.nn.relu(_bn_(h2 @ Wg2))
    h2 = jax.nn.relu(_bn_(h2 @ Wg3))
    ev = jax.vmap(_eig_feats_)(xyz)
    h3 = jax.nn.relu(ev @ We1 + be1) @ We2 + be2
    z = jnp.concatenate([h1, h2, h3], axis=-1)
    z = jax.nn.relu(_bn_(z @ Wc1))
    z = jax.nn.relu(_bn_(z @ Wc2))
    z = jax.nn.relu(_bn_(z @ Wc3))
    return xyz, jnp.transpose(z, (0, 2, 1))


# speedup vs baseline: 2.8846x; 2.8846x over previous
import functools
import math

import jax
import jax.numpy as jnp
from jax import lax
from jax.experimental import pallas as pl
from jax.experimental.pallas import tpu as pltpu

_F32 = jnp.float32
_DN_LAST = (((1,), (1,)), ((), ()))
_DN_STD = (((1,), (0,)), ((), ()))


def _dot_last(a, b):
    return lax.dot_general(a, b, _DN_LAST, preferred_element_type=_F32)


def _dot(a, b):
    return lax.dot_general(a, b, _DN_STD, preferred_element_type=_F32)


def _knn_body(ks, exact, x_ref, sqrow_ref, xbrow_ref, idx_ref, *out_refs):
    d_ref = out_refs[-1]
    a_refs = out_refs[:-1]
    x = x_ref[...]
    n = x.shape[0]
    sq = jnp.sum(x * x, axis=1, keepdims=True)
    if exact:
        xb = x.astype(jnp.bfloat16).astype(_F32)
        acc = xb[:, 0:1] * xbrow_ref[0:1, :]
        for c in range(1, x.shape[1]):
            acc = acc + xb[:, c:c + 1] * xbrow_ref[c:c + 1, :]
        xxt = acc
    else:
        xb = x.astype(jnp.bfloat16)
        xxt = _dot_last(xb, xb)
    d_ref[...] = sq - 2.0 * xxt + sqrow_ref[...]
    col = lax.broadcasted_iota(jnp.int32, (n, n), 1)
    kmax = max(ks)
    for j in range(kmax):
        d = d_ref[...]
        am = jnp.argmin(d, axis=1).astype(jnp.int32)
        idx_ref[:, j:j + 1] = am[:, None]
        d_ref[...] = jnp.where(col == am[:, None], jnp.inf, d)
        for ki, k in enumerate(ks):
            if j == k - 1:
                a_refs[ki][...] = jnp.where(jnp.isinf(d_ref[...]), 1.0, 0.0).astype(_F32)


def _knn(x, ks, exact=False):
    b, n, c = x.shape
    kmax = max(ks)
    sqrow = jnp.sum(x * x, axis=2)[:, None, :]
    xbrow = jnp.transpose(x.astype(jnp.bfloat16).astype(_F32), (0, 2, 1))
    kern = pl.pallas_call(
        functools.partial(_knn_body, ks, exact),
        grid=(b,),
        in_specs=[pl.BlockSpec((None, n, c), lambda i: (i, 0, 0)),
                  pl.BlockSpec((None, 1, n), lambda i: (i, 0, 0)),
                  pl.BlockSpec((None, c, n), lambda i: (i, 0, 0))],
        out_specs=[pl.BlockSpec((None, n, kmax), lambda i: (i, 0, 0))] +
                  [pl.BlockSpec((None, n, n), lambda i: (i, 0, 0)) for _ in ks],
        out_shape=[jax.ShapeDtypeStruct((b, n, kmax), jnp.int32)] +
                  [jax.ShapeDtypeStruct((b, n, n), _F32) for _ in ks],
        scratch_shapes=[pltpu.VMEM((n, n), _F32)],
    )
    outs = kern(x, sqrow, xbrow)
    return outs[0], outs[1:]


def _act(x, kind):
    if kind == "relu":
        return jnp.maximum(x, 0.0)
    if kind == "lrelu":
        return jnp.where(x >= 0.0, x, 0.2 * x)
    return x


def _mm_stats_body(x_ref, w_ref, y_ref, s1_ref, s2_ref):
    y = _dot(x_ref[...], w_ref[...])
    y_ref[...] = y
    s1_ref[...] = jnp.sum(y, axis=0, keepdims=True)
    s2_ref[...] = jnp.sum(y * y, axis=0, keepdims=True)


def _mm_stats(x, w):
    m, k = x.shape
    co = w.shape[1]
    return pl.pallas_call(
        _mm_stats_body,
        out_shape=[jax.ShapeDtypeStruct((m, co), _F32),
                   jax.ShapeDtypeStruct((1, co), _F32),
                   jax.ShapeDtypeStruct((1, co), _F32)],
    )(x, w)


def _bn_from_stats(s1, s2, cnt):
    mean = s1 / cnt
    var = jnp.maximum(s2 / cnt - mean * mean, 0.0)
    return mean, lax.rsqrt(var + 1e-5)


def _bnact_mm_stats_body(kind, cnt, y_ref, s1_ref, s2_ref, w_ref,
                         yn_ref, s1n_ref, s2n_ref):
    mean, scale = _bn_from_stats(s1_ref[...], s2_ref[...], cnt)
    xn = _act((y_ref[...] - mean) * scale, kind)
    yn = _dot(xn, w_ref[...])
    yn_ref[...] = yn
    s1n_ref[...] = jnp.sum(yn, axis=0, keepdims=True)
    s2n_ref[...] = jnp.sum(yn * yn, axis=0, keepdims=True)


def _bnact_mm_stats(y, s1, s2, w, kind, cnt):
    m = y.shape[0]
    co = w.shape[1]
    return pl.pallas_call(
        functools.partial(_bnact_mm_stats_body, kind, cnt),
        out_shape=[jax.ShapeDtypeStruct((m, co), _F32),
                   jax.ShapeDtypeStruct((1, co), _F32),
                   jax.ShapeDtypeStruct((1, co), _F32)],
    )(y, s1, s2, w)


def _bn_act_body(kind, cnt, y_ref, s1_ref, s2_ref, o_ref):
    mean, scale = _bn_from_stats(s1_ref[...], s2_ref[...], cnt)
    o_ref[...] = _act((y_ref[...] - mean) * scale, kind)


def _bn_act(y, s1, s2, kind, cnt):
    return pl.pallas_call(
        functools.partial(_bn_act_body, kind, cnt),
        out_shape=jax.ShapeDtypeStruct(y.shape, _F32),
    )(y, s1, s2)


def _eig_body(x_ref, xrow_ref, we1_ref, be1_ref, we2_ref, be2_ref, h3_ref):
    x = x_ref[...]
    n = x.shape[0]
    nf = float(n)
    df0 = x[:, 0:1] - xrow_ref[0:1, :]
    df1 = x[:, 1:2] - xrow_ref[1:2, :]
    df2 = x[:, 2:3] - xrow_ref[2:3, :]
    d2 = df0 * df0 + df1 * df1 + df2 * df2
    d = jnp.sqrt(d2 + 1e-12)
    ri = lax.broadcasted_iota(jnp.int32, (n, n), 0)
    ci = lax.broadcasted_iota(jnp.int32, (n, n), 1)
    eye = ri == ci
    max_d = jnp.max(jnp.where(eye, -jnp.inf, d))
    radius = max_d * 0.1
    maskf = jnp.where(jnp.where(eye, jnp.inf, d) < radius, 1.0, 0.0).astype(_F32)
    cnt = jnp.sum(maskf, axis=1, keepdims=True)
    cntc = jnp.maximum(cnt, 1.0)
    s1 = _dot(maskf, x)
    mean = s1 / cntc
    c0, c1, c2 = x[:, 0:1], x[:, 1:2], x[:, 2:3]
    cols = jnp.concatenate([c0 * c0, c0 * c1, c0 * c2, c1 * c1, c1 * c2, c2 * c2], axis=1)
    s2m = _dot(maskf, cols)
    m0, m1, m2 = mean[:, 0:1], mean[:, 1:2], mean[:, 2:3]
    a = s2m[:, 0:1] / nf - cnt * m0 * m0 / nf + 1e-6
    b = s2m[:, 3:4] / nf - cnt * m1 * m1 / nf + 2e-6
    c = s2m[:, 5:6] / nf - cnt * m2 * m2 / nf + 3e-6
    dd = s2m[:, 1:2] / nf - cnt * m0 * m1 / nf
    ee = s2m[:, 2:3] / nf - cnt * m0 * m2 / nf
    ff = s2m[:, 4:5] / nf - cnt * m1 * m2 / nf
    q = (a + b + c) / 3.0
    p1 = dd * dd + ee * ee + ff * ff
    p2 = (a - q) ** 2 + (b - q) ** 2 + (c - q) ** 2 + 2.0 * p1
    p = jnp.sqrt(p2 / 6.0)
    ps = jnp.maximum(p, 1e-12)
    b11, b22, b33 = (a - q) / ps, (b - q) / ps, (c - q) / ps
    bd, be, bf = dd / ps, ee / ps, ff / ps
    detb = (b11 * (b22 * b33 - bf * bf) - bd * (bd * b33 - bf * be)
            + be * (bd * bf - b22 * be))
    r = jnp.clip(detb * 0.5, -1.0, 1.0)
    phi = jnp.arctan2(jnp.sqrt(jnp.maximum(1.0 - r * r, 0.0)), r) / 3.0
    e1 = q + 2.0 * p * jnp.cos(phi)
    e3 = q + 2.0 * p * jnp.cos(phi + (2.0 * math.pi / 3.0))
    e2 = 3.0 * q - e1 - e3
    ev = jnp.concatenate([e3, e2, e1], axis=1)
    t = jnp.maximum(_dot(ev, we1_ref[...]) + be1_ref[...], 0.0)
    h3_ref[...] = _dot(t, we2_ref[...]) + be2_ref[...]


def _eig_h3(xyz, we1, be1, we2, be2):
    b, n, _ = xyz.shape
    xrow = jnp.transpose(xyz, (0, 2, 1))
    return pl.pallas_call(
        _eig_body,
        grid=(b,),
        in_specs=[pl.BlockSpec((None, n, 3), lambda i: (i, 0, 0)),
                  pl.BlockSpec((None, 3, n), lambda i: (i, 0, 0)),
                  pl.BlockSpec((3, 4), lambda i: (0, 0)),
                  pl.BlockSpec((1, 4), lambda i: (0, 0)),
                  pl.BlockSpec((4, 4), lambda i: (0, 0)),
                  pl.BlockSpec((1, 4), lambda i: (0, 0))],
        out_specs=pl.BlockSpec((None, n, 4), lambda i: (i, 0, 0)),
        out_shape=jax.ShapeDtypeStruct((b, n, 4), _F32),
    )(xyz, xrow, we1.reshape(3, 4), be1.reshape(1, 4), we2.reshape(4, 4), be2.reshape(1, 4))


def _bn_ref(x):
    axes = tuple(range(x.ndim - 1))
    m = jnp.mean(x, axis=axes, keepdims=True)
    v = jnp.var(x, axis=axes, keepdims=True)
    return (x - m) / jnp.sqrt(v + 1e-5)


def _edge_conv_jnp(x, idx, W):
    nb = jax.vmap(lambda xi, ii: xi[ii])(x, idx)
    center = x[:, :, None, :]
    feat = jnp.concatenate([nb - center, jnp.broadcast_to(center, nb.shape)], axis=-1)
    h = jax.nn.leaky_relu(_bn_ref(feat @ W), 0.2)
    return jnp.max(h, axis=2)


def kernel(pointcloud, W_sa1, W_sa2, Wd1, Wd2, Wd3, Wd4, Wd5, Wg1, Wg2, Wg3,
           We1, be1, We2, be2, Wc1, Wc2, Wc3, numpoints):
    xyz = pointcloud[..., 0:3]
    b, n, _ = xyz.shape
    m = b * n

    def _knn_jnp(x, k):
        sq = jnp.sum(x * x, axis=2)
        d = sq[:, :, None] - 2.0 * jnp.einsum('bnc,bmc->bnm', x, x) + sq[:, None, :]
        _, idx = jax.lax.top_k(-d, k)
        return idx, None
    idx32, _ = _knn(xyz, (20, 32), exact=True)
    idx20_0 = idx32[..., :20]

    nb = jax.vmap(lambda xi, ii: xi[ii])(xyz, idx32)
    rel = nb - xyz[:, :, None, :]
    h = jax.nn.relu(_bn_ref(rel @ W_sa1))
    h = jax.nn.relu(_bn_ref(h @ W_sa2))
    h1 = jnp.max(h, axis=2)

    x1 = _edge_conv_jnp(xyz, idx20_0, Wd1)
    idx1, _ = _knn(x1, (20,))
    x2 = _edge_conv_jnp(x1, idx1, Wd2)
    idx2, _ = _knn(x2, (20,))
    x3 = _edge_conv_jnp(x2, idx2, Wd3)
    idx3, _ = _knn(x3, (20,))
    x4 = _edge_conv_jnp(x3, idx3, Wd4)

    xc = jnp.concatenate([x1, x2, x3, x4], axis=-1).reshape(m, 512)
    y, s1, s2 = _mm_stats(xc, Wd5)
    y, s1, s2 = _bnact_mm_stats(y, s1, s2, Wg1, "lrelu", float(m))
    y, s1, s2 = _bnact_mm_stats(y, s1, s2, Wg2, "relu", float(m))
    y, s1, s2 = _bnact_mm_stats(y, s1, s2, Wg3, "relu", float(m))
    h2 = _bn_act(y, s1, s2, "relu", float(m))

    h3 = _eig_h3(xyz, We1, be1, We2, be2).reshape(m, 4)

    z = jnp.concatenate([h1.reshape(m, -1), h2, h3], axis=-1)
    y, s1, s2 = _mm_stats(z, Wc1)
    y, s1, s2 = _bnact_mm_stats(y, s1, s2, Wc2, "relu", float(m))
    y, s1, s2 = _bnact_mm_stats(y, s1, s2, Wc3, "relu", float(m))
    z = _bn_act(y, s1, s2, "relu", float(m)).reshape(b, n, -1)
    return xyz, jnp.transpose(z, (0, 2, 1))


# speedup vs baseline: 3.4723x; 1.2037x over previous
import functools
import math

import jax
import jax.numpy as jnp
from jax import lax
from jax.experimental import pallas as pl
from jax.experimental.pallas import tpu as pltpu
from jax.experimental.pallas import tpu_sc as plsc

_F32 = jnp.float32
_DN_LAST = (((1,), (1,)), ((), ()))
_DN_STD = (((1,), (0,)), ((), ()))


def _dot_last(a, b):
    return lax.dot_general(a, b, _DN_LAST, preferred_element_type=_F32)


def _dot(a, b):
    return lax.dot_general(a, b, _DN_STD, preferred_element_type=_F32)


def _knn_body(ks, exact, x_ref, sqrow_ref, xbrow_ref, idx_ref, *out_refs):
    d_ref = out_refs[-1]
    a_refs = out_refs[:-1]
    x = x_ref[...]
    n = x.shape[0]
    sq = jnp.sum(x * x, axis=1, keepdims=True)
    if exact:
        xb = x.astype(jnp.bfloat16).astype(_F32)
        acc = xb[:, 0:1] * xbrow_ref[0:1, :]
        for c in range(1, x.shape[1]):
            acc = acc + xb[:, c:c + 1] * xbrow_ref[c:c + 1, :]
        xxt = acc
    else:
        xb = x.astype(jnp.bfloat16)
        xxt = _dot_last(xb, xb)
    d_ref[...] = sq - 2.0 * xxt + sqrow_ref[...]
    col = lax.broadcasted_iota(jnp.int32, (n, n), 1)
    kmax = max(ks)
    for j in range(kmax):
        d = d_ref[...]
        am = jnp.argmin(d, axis=1).astype(jnp.int32)
        idx_ref[:, j:j + 1] = am[:, None]
        d_ref[...] = jnp.where(col == am[:, None], jnp.inf, d)
        for ki, k in enumerate(ks):
            if j == k - 1:
                a_refs[ki][...] = jnp.where(jnp.isinf(d_ref[...]), 1.0, 0.0).astype(_F32)


def _knn(x, ks, exact=False):
    b, n, c = x.shape
    kmax = max(ks)
    sqrow = jnp.sum(x * x, axis=2)[:, None, :]
    xbrow = jnp.transpose(x.astype(jnp.bfloat16).astype(_F32), (0, 2, 1))
    kern = pl.pallas_call(
        functools.partial(_knn_body, ks, exact),
        grid=(b,),
        in_specs=[pl.BlockSpec((None, n, c), lambda i: (i, 0, 0)),
                  pl.BlockSpec((None, 1, n), lambda i: (i, 0, 0)),
                  pl.BlockSpec((None, c, n), lambda i: (i, 0, 0))],
        out_specs=[pl.BlockSpec((None, n, kmax), lambda i: (i, 0, 0))] +
                  [pl.BlockSpec((None, n, n), lambda i: (i, 0, 0)) for _ in ks],
        out_shape=[jax.ShapeDtypeStruct((b, n, kmax), jnp.int32)] +
                  [jax.ShapeDtypeStruct((b, n, n), _F32) for _ in ks],
        scratch_shapes=[pltpu.VMEM((n, n), _F32)],
    )
    outs = kern(x, sqrow, xbrow)
    return outs[0], outs[1:]


def _act(x, kind):
    if kind == "relu":
        return jnp.maximum(x, 0.0)
    if kind == "lrelu":
        return jnp.where(x >= 0.0, x, 0.2 * x)
    return x


def _mm_stats_body(x_ref, w_ref, y_ref, s1_ref, s2_ref):
    y = _dot(x_ref[...], w_ref[...])
    y_ref[...] = y
    s1_ref[...] = jnp.sum(y, axis=0, keepdims=True)
    s2_ref[...] = jnp.sum(y * y, axis=0, keepdims=True)


def _mm_stats(x, w):
    m, k = x.shape
    co = w.shape[1]
    return pl.pallas_call(
        _mm_stats_body,
        out_shape=[jax.ShapeDtypeStruct((m, co), _F32),
                   jax.ShapeDtypeStruct((1, co), _F32),
                   jax.ShapeDtypeStruct((1, co), _F32)],
    )(x, w)


def _bn_from_stats(s1, s2, cnt):
    mean = s1 / cnt
    var = jnp.maximum(s2 / cnt - mean * mean, 0.0)
    return mean, lax.rsqrt(var + 1e-5)


def _bnact_mm_stats_body(kind, cnt, y_ref, s1_ref, s2_ref, w_ref,
                         yn_ref, s1n_ref, s2n_ref):
    mean, scale = _bn_from_stats(s1_ref[...], s2_ref[...], cnt)
    xn = _act((y_ref[...] - mean) * scale, kind)
    yn = _dot(xn, w_ref[...])
    yn_ref[...] = yn
    s1n_ref[...] = jnp.sum(yn, axis=0, keepdims=True)
    s2n_ref[...] = jnp.sum(yn * yn, axis=0, keepdims=True)


def _bnact_mm_stats(y, s1, s2, w, kind, cnt):
    m = y.shape[0]
    co = w.shape[1]
    return pl.pallas_call(
        functools.partial(_bnact_mm_stats_body, kind, cnt),
        out_shape=[jax.ShapeDtypeStruct((m, co), _F32),
                   jax.ShapeDtypeStruct((1, co), _F32),
                   jax.ShapeDtypeStruct((1, co), _F32)],
    )(y, s1, s2, w)


def _bn_act_body(kind, cnt, y_ref, s1_ref, s2_ref, o_ref):
    mean, scale = _bn_from_stats(s1_ref[...], s2_ref[...], cnt)
    o_ref[...] = _act((y_ref[...] - mean) * scale, kind)


def _bn_act(y, s1, s2, kind, cnt):
    return pl.pallas_call(
        functools.partial(_bn_act_body, kind, cnt),
        out_shape=jax.ShapeDtypeStruct(y.shape, _F32),
    )(y, s1, s2)


def _eig_body(x_ref, xrow_ref, we1_ref, be1_ref, we2_ref, be2_ref, h3_ref):
    x = x_ref[...]
    n = x.shape[0]
    nf = float(n)
    df0 = x[:, 0:1] - xrow_ref[0:1, :]
    df1 = x[:, 1:2] - xrow_ref[1:2, :]
    df2 = x[:, 2:3] - xrow_ref[2:3, :]
    d2 = df0 * df0 + df1 * df1 + df2 * df2
    d = jnp.sqrt(d2 + 1e-12)
    ri = lax.broadcasted_iota(jnp.int32, (n, n), 0)
    ci = lax.broadcasted_iota(jnp.int32, (n, n), 1)
    eye = ri == ci
    max_d = jnp.max(jnp.where(eye, -jnp.inf, d))
    radius = max_d * 0.1
    maskf = jnp.where(jnp.where(eye, jnp.inf, d) < radius, 1.0, 0.0).astype(_F32)
    cnt = jnp.sum(maskf, axis=1, keepdims=True)
    cntc = jnp.maximum(cnt, 1.0)
    s1 = _dot(maskf, x)
    mean = s1 / cntc
    c0, c1, c2 = x[:, 0:1], x[:, 1:2], x[:, 2:3]
    cols = jnp.concatenate([c0 * c0, c0 * c1, c0 * c2, c1 * c1, c1 * c2, c2 * c2], axis=1)
    s2m = _dot(maskf, cols)
    m0, m1, m2 = mean[:, 0:1], mean[:, 1:2], mean[:, 2:3]
    a = s2m[:, 0:1] / nf - cnt * m0 * m0 / nf + 1e-6
    b = s2m[:, 3:4] / nf - cnt * m1 * m1 / nf + 2e-6
    c = s2m[:, 5:6] / nf - cnt * m2 * m2 / nf + 3e-6
    dd = s2m[:, 1:2] / nf - cnt * m0 * m1 / nf
    ee = s2m[:, 2:3] / nf - cnt * m0 * m2 / nf
    ff = s2m[:, 4:5] / nf - cnt * m1 * m2 / nf
    q = (a + b + c) / 3.0
    p1 = dd * dd + ee * ee + ff * ff
    p2 = (a - q) ** 2 + (b - q) ** 2 + (c - q) ** 2 + 2.0 * p1
    p = jnp.sqrt(p2 / 6.0)
    ps = jnp.maximum(p, 1e-12)
    b11, b22, b33 = (a - q) / ps, (b - q) / ps, (c - q) / ps
    bd, be, bf = dd / ps, ee / ps, ff / ps
    detb = (b11 * (b22 * b33 - bf * bf) - bd * (bd * b33 - bf * be)
            + be * (bd * bf - b22 * be))
    r = jnp.clip(detb * 0.5, -1.0, 1.0)
    phi = jnp.arctan2(jnp.sqrt(jnp.maximum(1.0 - r * r, 0.0)), r) / 3.0
    e1 = q + 2.0 * p * jnp.cos(phi)
    e3 = q + 2.0 * p * jnp.cos(phi + (2.0 * math.pi / 3.0))
    e2 = 3.0 * q - e1 - e3
    ev = jnp.concatenate([e3, e2, e1], axis=1)
    t = jnp.maximum(_dot(ev, we1_ref[...]) + be1_ref[...], 0.0)
    h3_ref[...] = _dot(t, we2_ref[...]) + be2_ref[...]


def _eig_h3(xyz, we1, be1, we2, be2):
    b, n, _ = xyz.shape
    xrow = jnp.transpose(xyz, (0, 2, 1))
    return pl.pallas_call(
        _eig_body,
        grid=(b,),
        in_specs=[pl.BlockSpec((None, n, 3), lambda i: (i, 0, 0)),
                  pl.BlockSpec((None, 3, n), lambda i: (i, 0, 0)),
                  pl.BlockSpec((3, 4), lambda i: (0, 0)),
                  pl.BlockSpec((1, 4), lambda i: (0, 0)),
                  pl.BlockSpec((4, 4), lambda i: (0, 0)),
                  pl.BlockSpec((1, 4), lambda i: (0, 0))],
        out_specs=pl.BlockSpec((None, n, 4), lambda i: (i, 0, 0)),
        out_shape=jax.ShapeDtypeStruct((b, n, 4), _F32),
    )(xyz, xrow, we1.reshape(3, 4), be1.reshape(1, 4), we2.reshape(4, 4), be2.reshape(1, 4))


_NW = 32


def _sc_gather_reduce(xa, idxf, k, n_per_batch, co=None):
    mrows, ct = xa.shape
    c = ct if co is None else co
    p = mrows // _NW
    g = 4
    chunks = p // g
    w_per_b = n_per_batch // p
    mesh = plsc.VectorSubcoreMesh(core_axis_name="c", subcore_axis_name="s")

    @functools.partial(
        pl.kernel,
        out_type=[jax.ShapeDtypeStruct((mrows, c), _F32) for _ in range(3)],
        mesh=mesh,
        scratch_types=[
            pltpu.VMEM((g * k,), jnp.int32),
            pltpu.VMEM((g * k, ct), _F32),
            pltpu.VMEM((g, c), _F32),
            pltpu.VMEM((g, c), _F32),
            pltpu.VMEM((g, c), _F32),
            pltpu.SemaphoreType.DMA,
        ],
    )
    def kfn(xa_hbm, idx_hbm, gmax_hbm, gsum_hbm, gss_hbm,
            idx_v, rows_v, mx_v, sm_v, sq_v, sem):
        wid = lax.axis_index("s") * 2 + lax.axis_index("c")
        base = (wid // w_per_b) * n_per_batch

        def chunk(ci, carry):
            ib = wid * (p * k) + ci * (g * k)
            pltpu.sync_copy(idx_hbm.at[pl.ds(ib, g * k)], idx_v)
            for t in range(g * k // 16):
                sl = pl.ds(t * 16, 16)
                idx_v[sl] = idx_v[sl] + base
            pltpu.async_copy(xa_hbm.at[idx_v], rows_v, sem).wait()
            for gg in range(g):
                for t in range(c // 16):
                    sl = pl.ds(t * 16, 16)
                    v = rows_v[gg * k, sl]
                    mx = v
                    sm = v
                    sq = v * v
                    for j in range(1, k):
                        v = rows_v[gg * k + j, sl]
                        mx = jnp.maximum(mx, v)
                        sm = sm + v
                        sq = sq + v * v
                    mx_v[gg, sl] = mx
                    sm_v[gg, sl] = sm
                    sq_v[gg, sl] = sq
            pb = wid * p + ci * g
            pltpu.sync_copy(mx_v, gmax_hbm.at[pl.ds(pb, g)])
            pltpu.sync_copy(sm_v, gsum_hbm.at[pl.ds(pb, g)])
            pltpu.sync_copy(sq_v, gss_hbm.at[pl.ds(pb, g)])
            return carry

        lax.fori_loop(0, chunks, chunk, 0)

    return kfn(xa, idxf)


def _sc_gather_bn(xa, idxf, ms, k, n_per_batch, co=None):
    mrows, ct = xa.shape
    c = ct if co is None else co
    p = mrows // _NW
    g = 4
    chunks = p // g
    w_per_b = n_per_batch // p
    mesh = plsc.VectorSubcoreMesh(core_axis_name="c", subcore_axis_name="s")

    @functools.partial(
        pl.kernel,
        out_type=jax.ShapeDtypeStruct((mrows * k, c), _F32),
        mesh=mesh,
        scratch_types=[
            pltpu.VMEM((g * k,), jnp.int32),
            pltpu.VMEM((g * k, ct), _F32),
            pltpu.VMEM((g, ct), _F32),
            pltpu.VMEM((2, c), _F32),
            pltpu.VMEM((g * k, c), _F32),
            pltpu.SemaphoreType.DMA,
        ],
    )
    def kfn(xa_hbm, idx_hbm, ms_hbm, q_hbm,
            idx_v, rows_v, cent_v, ms_v, qb_v, sem):
        wid = lax.axis_index("s") * 2 + lax.axis_index("c")
        base = (wid // w_per_b) * n_per_batch
        pltpu.sync_copy(ms_hbm, ms_v)

        def chunk(ci, carry):
            ib = wid * (p * k) + ci * (g * k)
            pb = wid * p + ci * g
            pltpu.sync_copy(idx_hbm.at[pl.ds(ib, g * k)], idx_v)
            for t in range(g * k // 16):
                sl = pl.ds(t * 16, 16)
                idx_v[sl] = idx_v[sl] + base
            pltpu.async_copy(xa_hbm.at[idx_v], rows_v, sem).wait()
            pltpu.sync_copy(xa_hbm.at[pl.ds(pb, g)], cent_v)
            for t in range(c // 16):
                sl = pl.ds(t * 16, 16)
                mvec = ms_v[0, sl]
                svec = ms_v[1, sl]
                for gg in range(g):
                    cv = cent_v[gg, sl] + mvec
                    for j in range(k):
                        v = rows_v[gg * k + j, sl]
                        qb_v[gg * k + j, sl] = jnp.maximum((v - cv) * svec, 0.0)
            pltpu.sync_copy(qb_v, q_hbm.at[pl.ds(pb * k, g * k)])
            return carry

        lax.fori_loop(0, chunks, chunk, 0)

    return kfn(xa, idxf, ms)


def _ec_finish_body(k, gmax_ref, gsum_ref, gss_ref, xct_ref, o_ref):
    xct = xct_ref[...]
    gsum = gsum_ref[...]
    kf = float(k)
    cnt = float(gsum.shape[0] * k)
    s1 = jnp.sum(gsum + kf * xct, axis=0, keepdims=True)
    s2 = jnp.sum(gss_ref[...] + 2.0 * xct * gsum + kf * xct * xct, axis=0, keepdims=True)
    mean = s1 / cnt
    var = jnp.maximum(s2 / cnt - mean * mean, 0.0)
    scale = lax.rsqrt(var + 1e-5)
    o_ref[...] = _act((gmax_ref[...] + xct - mean) * scale, "lrelu")


def _ec_finish(gmax, gsum, gss, xct, k):
    return pl.pallas_call(
        functools.partial(_ec_finish_body, k),
        out_shape=jax.ShapeDtypeStruct(gmax.shape, _F32),
    )(gmax, gsum, gss, xct)


def _sa_stats_body(k, gsum_ref, gss_ref, xa_ref, ms_ref):
    xa = xa_ref[...]
    gsum = gsum_ref[...]
    kf = float(k)
    cnt = float(xa.shape[0] * k)
    s1 = jnp.sum(gsum - kf * xa, axis=0, keepdims=True)
    s2 = jnp.sum(gss_ref[...] - 2.0 * xa * gsum + kf * xa * xa, axis=0, keepdims=True)
    mean = s1 / cnt
    var = jnp.maximum(s2 / cnt - mean * mean, 0.0)
    scale = lax.rsqrt(var + 1e-5)
    ms_ref[0:1, :] = mean
    ms_ref[1:2, :] = scale


def _sa_stats(gsum, gss, xa, k):
    return pl.pallas_call(
        functools.partial(_sa_stats_body, k),
        out_shape=jax.ShapeDtypeStruct((2, xa.shape[1]), _F32),
    )(gsum, gss, xa)


def _mm_groupmax_body(group, rows, x_ref, w_ref, ym_ref, s1_ref, s2_ref):
    i = pl.program_id(0)
    y = _dot(x_ref[...], w_ref[...])

    @pl.when(i == 0)
    def _init():
        s1_ref[...] = jnp.zeros(s1_ref.shape, _F32)
        s2_ref[...] = jnp.zeros(s2_ref.shape, _F32)

    s1_ref[...] += jnp.sum(y, axis=0, keepdims=True)
    s2_ref[...] += jnp.sum(y * y, axis=0, keepdims=True)
    for g in range(rows // group):
        ym_ref[g:g + 1, :] = jnp.max(y[g * group:(g + 1) * group, :], axis=0,
                                     keepdims=True)


def _mm_groupmax_stats(x, w, group, blocks=32):
    mrows, kdim = x.shape
    co = w.shape[1]
    rows = mrows // blocks
    pts = rows // group
    kern = pl.pallas_call(
        functools.partial(_mm_groupmax_body, group, rows),
        grid=(blocks,),
        in_specs=[pl.BlockSpec((rows, kdim), lambda i: (i, 0)),
                  pl.BlockSpec((kdim, co), lambda i: (0, 0))],
        out_specs=[pl.BlockSpec((pts, co), lambda i: (i, 0)),
                   pl.BlockSpec((1, co), lambda i: (0, 0)),
                   pl.BlockSpec((1, co), lambda i: (0, 0))],
        out_shape=[jax.ShapeDtypeStruct((mrows // group, co), _F32),
                   jax.ShapeDtypeStruct((1, co), _F32),
                   jax.ShapeDtypeStruct((1, co), _F32)],
    )
    return kern(x, w)


def _bn_ref(x):
    axes = tuple(range(x.ndim - 1))
    m = jnp.mean(x, axis=axes, keepdims=True)
    v = jnp.var(x, axis=axes, keepdims=True)
    return (x - m) / jnp.sqrt(v + 1e-5)


def _edge_conv_jnp(x, idx, W):
    nb = jax.vmap(lambda xi, ii: xi[ii])(x, idx)
    center = x[:, :, None, :]
    feat = jnp.concatenate([nb - center, jnp.broadcast_to(center, nb.shape)], axis=-1)
    h = jax.nn.leaky_relu(_bn_ref(feat @ W), 0.2)
    return jnp.max(h, axis=2)


def kernel(pointcloud, W_sa1, W_sa2, Wd1, Wd2, Wd3, Wd4, Wd5, Wg1, Wg2, Wg3,
           We1, be1, We2, be2, Wc1, Wc2, Wc3, numpoints):
    xyz = pointcloud[..., 0:3]
    b, n, _ = xyz.shape
    m = b * n

    idx32, _ = _knn(xyz, (20, 32), exact=True)
    idx20_0 = idx32[..., :20]
    xyz2d = xyz.reshape(m, 3)

    def _edge_conv_sc(x2d, idx, W):
        cin = x2d.shape[1]
        co = W.shape[1]
        ct = max(co, 128)
        pad = [jnp.zeros((cin, ct - co), _F32)] if ct > co else []
        wcat = jnp.concatenate([W[:cin]] + pad + [W[cin:] - W[:cin]], axis=1)
        y, _, _ = _mm_stats(x2d, wcat)
        xa, xct = y[:, :ct], y[:, ct:]
        k = idx.shape[-1]
        gmax, gsum, gss = _sc_gather_reduce(xa, idx.reshape(-1), k, n, co)
        return _ec_finish(gmax, gsum, gss, xct, k)

    nb = jax.vmap(lambda xi, ii: xi[ii])(xyz, idx32)
    rel = nb - xyz[:, :, None, :]
    hh = jax.nn.relu(_bn_ref(rel @ W_sa1))
    hh = jax.nn.relu(_bn_ref(hh @ W_sa2))
    h1 = jnp.max(hh, axis=2).reshape(m, -1)

    x1 = _edge_conv_jnp(xyz, idx20_0, Wd1)
    idx1, _ = _knn(x1, (20,))
    x2 = _edge_conv_jnp(x1, idx1, Wd2)
    idx2, _ = _knn(x2, (20,))
    x3 = _edge_conv_jnp(x2, idx2, Wd3)
    idx3, _ = _knn(x3, (20,))
    x4 = _edge_conv_sc(x3.reshape(m, -1), idx3, Wd4)
    x1, x2, x3 = x1.reshape(m, -1), x2.reshape(m, -1), x3.reshape(m, -1)

    xc = jnp.concatenate([x1, x2, x3, x4], axis=-1)
    y, s1, s2 = _mm_stats(xc, Wd5)
    y, s1, s2 = _bnact_mm_stats(y, s1, s2, Wg1, "lrelu", float(m))
    y, s1, s2 = _bnact_mm_stats(y, s1, s2, Wg2, "relu", float(m))
    y, s1, s2 = _bnact_mm_stats(y, s1, s2, Wg3, "relu", float(m))
    h2 = _bn_act(y, s1, s2, "relu", float(m))

    h3 = _eig_h3(xyz, We1, be1, We2, be2).reshape(m, 4)

    z = jnp.concatenate([h1, h2, h3], axis=-1)
    y, s1, s2 = _mm_stats(z, Wc1)
    y, s1, s2 = _bnact_mm_stats(y, s1, s2, Wc2, "relu", float(m))
    y, s1, s2 = _bnact_mm_stats(y, s1, s2, Wc3, "relu", float(m))
    z = _bn_act(y, s1, s2, "relu", float(m)).reshape(b, n, -1)
    return xyz, jnp.transpose(z, (0, 2, 1))
